# Initial kernel scaffold; baseline (speedup 1.0000x reference)
import jax
import jax.numpy as jnp
from jax.experimental import pallas as pl


def kernel(x, emb):
    raise NotImplementedError("write your pallas kernel here")


# speedup vs baseline: 4.4235x; 4.4235x over previous
import functools

import jax
import jax.numpy as jnp
from jax import lax
from jax.experimental import pallas as pl
from jax.experimental.pallas import tpu as pltpu
from jax.experimental.pallas import tpu_sc as plsc

N = 100000
F = 9
V = 100
H = 128
L = 16
C = 80
NW = 32
NCHUNK = N // C


def _sc_body(idx_hbm, table_hbm, out_hbm, idx_t_v, bufs_v, out_v,
             idx_sem, gather_sem):
    cid = lax.axis_index("c")
    sid = lax.axis_index("s")
    wid = sid * 2 + cid

    def chunk_step(j, carry):
        chunk = wid + j * NW
        base = chunk * C

        icps = [
            pltpu.async_copy(idx_hbm.at[pl.ds(i * N + base, C)],
                             idx_t_v.at[i], idx_sem)
            for i in range(F)
        ]
        for cp in icps:
            cp.wait()

        cps = [
            pltpu.async_copy(table_hbm.at[idx_t_v.at[i]], bufs_v.at[i],
                             gather_sem)
            for i in range(F)
        ]
        for cp in cps:
            cp.wait()

        def row_step(r, c2):
            for cc in range(H // L):
                acc = bufs_v[0, r, pl.ds(cc * L, L)]
                for i in range(1, F):
                    acc = acc + bufs_v[i, r, pl.ds(cc * L, L)]
                out_v[r, pl.ds(cc * L, L)] = acc
            return c2

        lax.fori_loop(0, C, row_step, 0)

        pltpu.sync_copy(out_v, out_hbm.at[pl.ds(base, C)])
        return carry

    nj = (NCHUNK - wid + NW - 1) // NW
    lax.fori_loop(0, nj, chunk_step, 0)


@functools.lru_cache(maxsize=1)
def _build_encoder():
    @functools.partial(
        pl.kernel,
        out_type=jax.ShapeDtypeStruct((N, H), jnp.float32),
        mesh=plsc.VectorSubcoreMesh(core_axis_name="c", subcore_axis_name="s"),
        scratch_types=[
            pltpu.VMEM((F, C), jnp.int32),
            pltpu.VMEM((F, C, H), jnp.float32),
            pltpu.VMEM((C, H), jnp.float32),
            pltpu.SemaphoreType.DMA,
            pltpu.SemaphoreType.DMA,
        ],
    )
    def _sc_encoder(idx_hbm, table_hbm, out_hbm, idx_t_v, bufs_v,
                    out_v, idx_sem, gather_sem):
        _sc_body(idx_hbm, table_hbm, out_hbm, idx_t_v, bufs_v, out_v,
                 idx_sem, gather_sem)

    return _sc_encoder


def kernel(x, emb):
    table = emb.reshape(F * V, H)
    idx = x.T.astype(jnp.int32) + (jnp.arange(F, dtype=jnp.int32) * V)[:, None]
    return _build_encoder()(idx.reshape(F * N), table)


# speedup vs baseline: 4.9282x; 1.1141x over previous
import functools

import jax
import jax.numpy as jnp
from jax import lax
from jax.experimental import pallas as pl
from jax.experimental.pallas import tpu as pltpu
from jax.experimental.pallas import tpu_sc as plsc

N = 100000
F = 9
V = 100
H = 128
L = 16
C = 80
NW = 32
NCHUNK = N // C


def _sc_body(idx_hbm, table_hbm, out_hbm, idx_t_v, out_v, idx_sem,
             gather_sem):
    cid = lax.axis_index("c")
    sid = lax.axis_index("s")
    wid = sid * 2 + cid

    def chunk_step(j, carry):
        chunk = wid + j * NW
        base = chunk * C

        icps = [
            pltpu.async_copy(idx_hbm.at[pl.ds(i * N + base, C)],
                             idx_t_v.at[i], idx_sem)
            for i in range(F)
        ]
        for cp in icps:
            cp.wait()

        g0 = pltpu.async_copy(table_hbm.at[idx_t_v.at[0]], out_v, gather_sem)
        g0.wait()
        cps = [
            pltpu.async_copy(table_hbm.at[idx_t_v.at[i]], out_v, gather_sem,
                             add=True)
            for i in range(1, F)
        ]
        for cp in cps:
            cp.wait()

        pltpu.sync_copy(out_v, out_hbm.at[pl.ds(base, C)])
        return carry

    nj = (NCHUNK - wid + NW - 1) // NW
    lax.fori_loop(0, nj, chunk_step, 0)


@functools.lru_cache(maxsize=1)
def _build_encoder():
    @functools.partial(
        pl.kernel,
        out_type=jax.ShapeDtypeStruct((N, H), jnp.float32),
        mesh=plsc.VectorSubcoreMesh(core_axis_name="c", subcore_axis_name="s"),
        scratch_types=[
            pltpu.VMEM((F, C), jnp.int32),
            pltpu.VMEM((C, H), jnp.float32),
            pltpu.SemaphoreType.DMA,
            pltpu.SemaphoreType.DMA,
        ],
    )
    def _sc_encoder(idx_hbm, table_hbm, out_hbm, idx_t_v, out_v, idx_sem,
                    gather_sem):
        _sc_body(idx_hbm, table_hbm, out_hbm, idx_t_v, out_v, idx_sem,
                 gather_sem)

    return _sc_encoder


def kernel(x, emb):
    table = emb.reshape(F * V, H)
    idx = x.T.astype(jnp.int32) + (jnp.arange(F, dtype=jnp.int32) * V)[:, None]
    return _build_encoder()(idx.reshape(F * N), table)
